# Initial kernel scaffold; baseline (speedup 1.0000x reference)
import jax
import jax.numpy as jnp
from jax.experimental import pallas as pl


def kernel(X, E, Ea, W1, as1, ad1, b1, W2, as2, ad2, b2, W3, as3, ad3, b3, W4, as4, ad4, b4):
    raise NotImplementedError("write your pallas kernel here")


# speedup vs baseline: 26.4482x; 26.4482x over previous
import functools

import jax
import jax.numpy as jnp
from jax import lax
from jax.experimental import pallas as pl
from jax.experimental.pallas import tpu as pltpu
from jax.experimental.pallas import tpu_sc as plsc

N = 10000
EDGES = 160000
HEADS = 8
C = 32
F = 256
HALF = 128
ROW = 144
BN = 1000
GRID = N // BN
W = 40
NWIN = 250
NWPAD = 256
EPT = EDGES // 16
NPAD = 10240
RPT = NPAD // 16

_HIGH = jax.lax.Precision.HIGHEST


def _dense_tail(x, w_ref, as_ref, ad_ref, hx0o, hx1o, at0o, at1o, moo):
    h = jnp.dot(x, w_ref[...], preferred_element_type=jnp.float32,
                precision=_HIGH)
    h3 = h.reshape(BN, HEADS, C)
    a_s = jnp.sum(h3 * as_ref[...][None], axis=-1)
    a_d = jnp.sum(h3 * ad_ref[...][None], axis=-1)
    pad = jnp.zeros((BN, ROW - HALF - 4), jnp.float32)
    hx0o[...] = jnp.concatenate([h[:, :HALF], a_s[:, :4], pad], axis=1)
    hx1o[...] = jnp.concatenate([h[:, HALF:], a_s[:, 4:], pad], axis=1)
    padd = jnp.zeros((BN, 12), jnp.float32)
    at0o[...] = jnp.concatenate([a_d[:, :4], padd], axis=1)
    at1o[...] = jnp.concatenate([a_d[:, 4:], padd], axis=1)
    blockm = jnp.stack([jnp.max(a_s, axis=0), jnp.max(a_d, axis=0)])

    @pl.when(pl.program_id(0) == 0)
    def _():
        moo[...] = jnp.full((2, HEADS), -1e30, jnp.float32)

    moo[...] = jnp.maximum(moo[...], blockm)


def _tc_first(x_ref, w_ref, as_ref, ad_ref, hx0o, hx1o, at0o, at1o, moo):
    _dense_tail(x_ref[...], w_ref, as_ref, ad_ref,
                hx0o, hx1o, at0o, at1o, moo)


def _normalize_half(o, hxp, atp, mh, bh):
    hh = hxp[:, :HALF]
    asv = hxp[:, HALF:HALF + 4]
    adv = atp[:, :4]
    pre = asv + adv
    ev = jnp.where(pre > 0.0, pre, 0.2 * pre)
    exs = jnp.exp(ev - mh[None, :])
    exb = jnp.reshape(jnp.broadcast_to(exs[:, :, None], (BN, 4, C)),
                      (BN, HALF))
    num = o[:, :HALF] + exb * hh
    den = o[:, HALF:HALF + 4] + exs + 1e-16
    denb = jnp.reshape(jnp.broadcast_to(den[:, :, None], (BN, 4, C)),
                       (BN, HALF))
    return num / denb + bh[None, :]


def _tc_mid(o0, o1, hx0, hx1, at0, at1, mv, bp, w_ref, as_ref, ad_ref,
            hx0o, hx1o, at0o, at1o, moo, xo):
    mvv = mv[0]
    x0 = _normalize_half(o0[...], hx0[...], at0, mvv[:4], bp[0, :HALF])
    x1 = _normalize_half(o1[...], hx1[...], at1, mvv[4:], bp[0, HALF:])
    x = jnp.concatenate([x0, x1], axis=1)
    xo[...] = x
    _dense_tail(x, w_ref, as_ref, ad_ref, hx0o, hx1o, at0o, at1o, moo)


def _tc_last(o0, o1, hx0, hx1, at0, at1, mv, bp, xo):
    mvv = mv[0]
    x0 = _normalize_half(o0[...], hx0[...], at0, mvv[:4], bp[0, :HALF])
    x1 = _normalize_half(o1[...], hx1[...], at1, mvv[4:], bp[0, HALF:])
    xo[...] = jnp.concatenate([x0, x1], axis=1)


def _sc_edge_body(hx0, hx1, at0, at1, src_h, dst2_h, m_h,
                  out0_h, out1_h,
                  srcv, dst2v, gbuf, cbuf, adbuf, mvr, sem, sem2, acc):
    c = lax.axis_index("c")
    s = lax.axis_index("s")
    z16 = jnp.zeros((16,), jnp.float32)

    pltpu.sync_copy(src_h.at[pl.ds(s * EPT, EPT)], srcv)
    pltpu.sync_copy(dst2_h.at[pl.ds(s * NWPAD, NWPAD)], dst2v)
    pltpu.sync_copy(m_h.at[pl.ds(c * 16, 16)], mvr)

    def zb(i, _):
        gbuf[i // 9, pl.ds((i % 9) * 16, 16)] = z16
        return _
    lax.fori_loop(0, W * 9, zb, None)

    def zs(j, _):
        pltpu.sync_copy(gbuf, acc.at[pl.ds(s * RPT + j * W, W)])
        return _
    lax.fori_loop(0, RPT // W, zs, None)

    plsc.subcore_barrier()

    lanes = lax.broadcasted_iota(jnp.int32, (16,), 0)
    hmask = lanes < 4
    mvec = jnp.where(hmask, mvr[...], 0.0)

    def win(w, _):
        isl = srcv.at[pl.ds(w * W, W)]
        dsl = dst2v.at[w]

        @pl.when(c == 0)
        def _():
            pltpu.async_copy(hx0.at[isl], gbuf, sem).wait()
            pltpu.async_copy(at0.at[dsl], adbuf, sem2).wait()

        @pl.when(c == 1)
        def _():
            pltpu.async_copy(hx1.at[isl], gbuf, sem).wait()
            pltpu.async_copy(at1.at[dsl], adbuf, sem2).wait()

        def edge(r, _):
            asv = gbuf[r, pl.ds(HALF, 16)]
            adv = adbuf[r, pl.ds(0, 16)]
            pre = asv + adv
            ev = jnp.where(pre > 0.0, pre, 0.2 * pre)
            exv = jnp.where(hmask, jnp.exp(ev - mvec), 0.0)
            cbuf[r, pl.ds(HALF, 16)] = exv
            for hh in range(4):
                ebc = jnp.full((16,), exv[hh], jnp.float32)
                for k in (2 * hh, 2 * hh + 1):
                    cbuf[r, pl.ds(k * 16, 16)] = (
                        gbuf[r, pl.ds(k * 16, 16)] * ebc)
            return _

        lax.fori_loop(0, W, edge, None)
        pltpu.sync_copy(cbuf, acc.at[dsl], add=True)
        return _

    lax.fori_loop(0, NWIN, win, None)
    plsc.subcore_barrier()

    def rb(j, _):
        sl = pl.ds(s * RPT + j * W, W)

        @pl.when(c == 0)
        def _():
            pltpu.sync_copy(acc.at[sl], out0_h.at[sl])

        @pl.when(c == 1)
        def _():
            pltpu.sync_copy(acc.at[sl], out1_h.at[sl])
        return _
    lax.fori_loop(0, RPT // W, rb, None)


_sc_edge = functools.partial(
    pl.kernel,
    mesh=plsc.VectorSubcoreMesh(core_axis_name="c", subcore_axis_name="s"),
    compiler_params=pltpu.CompilerParams(use_tc_tiling_on_sc=False),
    out_type=[jax.ShapeDtypeStruct((NPAD, ROW), jnp.float32),
              jax.ShapeDtypeStruct((NPAD, ROW), jnp.float32)],
    scratch_types=[
        pltpu.VMEM((EPT,), jnp.int32),
        pltpu.VMEM((NWPAD, W), jnp.int32),
        pltpu.VMEM((W, ROW), jnp.float32),
        pltpu.VMEM((W, ROW), jnp.float32),
        pltpu.VMEM((W, 16), jnp.float32),
        pltpu.VMEM((16,), jnp.float32),
        pltpu.SemaphoreType.DMA,
        pltpu.SemaphoreType.DMA,
        pltpu.VMEM_SHARED((NPAD, ROW), jnp.float32),
    ],
)(_sc_edge_body)


def _row_specs(n_args):
    return [pl.BlockSpec((BN, ROW), lambda i: (i, 0)) for _ in range(n_args)]


def _tc_outs():
    return (
        [jax.ShapeDtypeStruct((N, ROW), jnp.float32),
         jax.ShapeDtypeStruct((N, ROW), jnp.float32),
         jax.ShapeDtypeStruct((N, 16), jnp.float32),
         jax.ShapeDtypeStruct((N, 16), jnp.float32),
         jax.ShapeDtypeStruct((2, HEADS), jnp.float32)],
        [pl.BlockSpec((BN, ROW), lambda i: (i, 0)),
         pl.BlockSpec((BN, ROW), lambda i: (i, 0)),
         pl.BlockSpec((BN, 16), lambda i: (i, 0)),
         pl.BlockSpec((BN, 16), lambda i: (i, 0)),
         pl.BlockSpec((2, HEADS), lambda i: (0, 0))],
    )


def _full(shape):
    return pl.BlockSpec(shape, lambda i: tuple(0 for _ in shape))


def kernel(X, E, Ea, W1, as1, ad1, b1, W2, as2, ad2, b2,
           W3, as3, ad3, b3, W4, as4, ad4, b4):
    del Ea
    src = E[:, 0].astype(jnp.int32)
    dst = E[:, 1].astype(jnp.int32)
    dst2 = jnp.pad(dst.reshape(16, NWIN, W),
                   ((0, 0), (0, NWPAD - NWIN), (0, 0))).reshape(16 * NWPAD, W)

    atts = [(as1.reshape(HEADS, C), ad1.reshape(HEADS, C)),
            (as2.reshape(HEADS, C), ad2.reshape(HEADS, C)),
            (as3.reshape(HEADS, C), ad3.reshape(HEADS, C)),
            (as4.reshape(HEADS, C), ad4.reshape(HEADS, C))]
    Ws = [W1, W2, W3, W4]
    bs = [b1.reshape(1, F), b2.reshape(1, F), b3.reshape(1, F),
          b4.reshape(1, F)]

    outs, out_specs = _tc_outs()
    first = pl.pallas_call(
        _tc_first,
        grid=(GRID,),
        in_specs=[pl.BlockSpec((BN, 36), lambda i: (i, 0)),
                  _full((36, F)), _full((HEADS, C)), _full((HEADS, C))],
        out_specs=out_specs,
        out_shape=outs,
    )
    hx0, hx1, at0, at1, mo = first(X, W1, atts[0][0], atts[0][1])

    def m_vecs(mo):
        mv = jnp.maximum(mo[0] + mo[1], 0.0)
        m16 = jnp.concatenate([jnp.tile(mv[:4], 4), jnp.tile(mv[4:], 4)])
        return mv.reshape(1, HEADS), m16

    x3 = None
    for li in (1, 2, 3):
        mv, m16 = m_vecs(mo)
        agg0, agg1 = _sc_edge(hx0, hx1, at0, at1, src, dst2, m16)
        outs, out_specs = _tc_outs()
        mid = pl.pallas_call(
            _tc_mid,
            grid=(GRID,),
            in_specs=([pl.BlockSpec((BN, ROW), lambda i: (i, 0)),
                       pl.BlockSpec((BN, ROW), lambda i: (i, 0)),
                       pl.BlockSpec((BN, ROW), lambda i: (i, 0)),
                       pl.BlockSpec((BN, ROW), lambda i: (i, 0)),
                       pl.BlockSpec((BN, 16), lambda i: (i, 0)),
                       pl.BlockSpec((BN, 16), lambda i: (i, 0)),
                       _full((1, HEADS)), _full((1, F)),
                       _full((F, F)), _full((HEADS, C)), _full((HEADS, C))]),
            out_specs=out_specs + [pl.BlockSpec((BN, F), lambda i: (i, 0))],
            out_shape=outs + [jax.ShapeDtypeStruct((N, F), jnp.float32)],
        )
        hx0, hx1, at0, at1, mo, x = mid(
            agg0, agg1, hx0, hx1, at0, at1, mv, bs[li - 1],
            Ws[li], atts[li][0], atts[li][1])
        if li == 3:
            x3 = x

    mv, m16 = m_vecs(mo)
    agg0, agg1 = _sc_edge(hx0, hx1, at0, at1, src, dst2, m16)
    last = pl.pallas_call(
        _tc_last,
        grid=(GRID,),
        in_specs=([pl.BlockSpec((BN, ROW), lambda i: (i, 0)),
                   pl.BlockSpec((BN, ROW), lambda i: (i, 0)),
                   pl.BlockSpec((BN, ROW), lambda i: (i, 0)),
                   pl.BlockSpec((BN, ROW), lambda i: (i, 0)),
                   pl.BlockSpec((BN, 16), lambda i: (i, 0)),
                   pl.BlockSpec((BN, 16), lambda i: (i, 0)),
                   _full((1, HEADS)), _full((1, F))]),
        out_specs=pl.BlockSpec((BN, F), lambda i: (i, 0)),
        out_shape=jax.ShapeDtypeStruct((N, F), jnp.float32),
    )
    x4 = last(agg0, agg1, hx0, hx1, at0, at1, mv, bs[3])
    return (x3, x4)


# speedup vs baseline: 46.6420x; 1.7635x over previous
import functools

import jax
import jax.numpy as jnp
from jax import lax
from jax.experimental import pallas as pl
from jax.experimental.pallas import tpu as pltpu
from jax.experimental.pallas import tpu_sc as plsc

N = 10000
EDGES = 160000
HEADS = 8
C = 32
F = 256
HALF = 128
ROW = 144
BN = 1000
GRID = N // BN
W = 40
NWIN = 250
NWPAD = 256
EPT = EDGES // 16
NPAD = 10240
RPT = NPAD // 16

_HIGH = jax.lax.Precision.HIGHEST


def _dense_tail(x, w_ref, as_ref, ad_ref, hx0o, hx1o, at0o, at1o, moo):
    h = jnp.dot(x, w_ref[...], preferred_element_type=jnp.float32,
                precision=_HIGH)
    h3 = h.reshape(BN, HEADS, C)
    a_s = jnp.sum(h3 * as_ref[...][None], axis=-1)
    a_d = jnp.sum(h3 * ad_ref[...][None], axis=-1)
    pad = jnp.zeros((BN, ROW - HALF - 4), jnp.float32)
    hx0o[...] = jnp.concatenate([h[:, :HALF], a_s[:, :4], pad], axis=1)
    hx1o[...] = jnp.concatenate([h[:, HALF:], a_s[:, 4:], pad], axis=1)
    padd = jnp.zeros((BN, 12), jnp.float32)
    at0o[...] = jnp.concatenate([a_d[:, :4], padd], axis=1)
    at1o[...] = jnp.concatenate([a_d[:, 4:], padd], axis=1)
    blockm = jnp.stack([jnp.max(a_s, axis=0), jnp.max(a_d, axis=0)])

    @pl.when(pl.program_id(0) == 0)
    def _():
        moo[...] = jnp.full((2, HEADS), -1e30, jnp.float32)

    moo[...] = jnp.maximum(moo[...], blockm)


def _tc_first(x_ref, w_ref, as_ref, ad_ref, hx0o, hx1o, at0o, at1o, moo):
    _dense_tail(x_ref[...], w_ref, as_ref, ad_ref,
                hx0o, hx1o, at0o, at1o, moo)


def _normalize_half(o, hxp, atp, mh, bh):
    hh = hxp[:, :HALF]
    asv = hxp[:, HALF:HALF + 4]
    adv = atp[:, :4]
    pre = asv + adv
    ev = jnp.where(pre > 0.0, pre, 0.2 * pre)
    exs = jnp.exp(ev - mh[None, :])
    exb = jnp.reshape(jnp.broadcast_to(exs[:, :, None], (BN, 4, C)),
                      (BN, HALF))
    num = o[:, :HALF] + exb * hh
    den = o[:, HALF:HALF + 4] + exs + 1e-16
    denb = jnp.reshape(jnp.broadcast_to(den[:, :, None], (BN, 4, C)),
                       (BN, HALF))
    return num / denb + bh[None, :]


def _tc_mid(o0, o1, hx0, hx1, at0, at1, mv, bp, w_ref, as_ref, ad_ref,
            hx0o, hx1o, at0o, at1o, moo, xo):
    mvv = mv[0]
    x0 = _normalize_half(o0[...], hx0[...], at0, mvv[:4], bp[0, :HALF])
    x1 = _normalize_half(o1[...], hx1[...], at1, mvv[4:], bp[0, HALF:])
    x = jnp.concatenate([x0, x1], axis=1)
    xo[...] = x
    _dense_tail(x, w_ref, as_ref, ad_ref, hx0o, hx1o, at0o, at1o, moo)


def _tc_last(o0, o1, hx0, hx1, at0, at1, mv, bp, xo):
    mvv = mv[0]
    x0 = _normalize_half(o0[...], hx0[...], at0, mvv[:4], bp[0, :HALF])
    x1 = _normalize_half(o1[...], hx1[...], at1, mvv[4:], bp[0, HALF:])
    xo[...] = jnp.concatenate([x0, x1], axis=1)


def _sc_edge_body(hx0, hx1, at0, at1, src2_h, dst2_h, m_h,
                  out0_h, out1_h,
                  src2v, dst2v, g0, g1, cb, a0, a1, mvr,
                  sg0, sg1, sa0, sa1, ss, acc):
    c = lax.axis_index("c")
    s = lax.axis_index("s")
    z16 = jnp.zeros((16,), jnp.float32)
    GB, AB = (g0, g1), (a0, a1)
    SG, SA = (sg0, sg1), (sa0, sa1)

    pltpu.sync_copy(src2_h.at[pl.ds(s * NWPAD, NWIN)], src2v)
    pltpu.sync_copy(dst2_h.at[pl.ds(s * NWPAD, NWIN)], dst2v)
    pltpu.sync_copy(m_h.at[pl.ds(c * 16, 16)], mvr)

    def zb(i, _):
        g0[i // 9, pl.ds((i % 9) * 16, 16)] = z16
        return _
    lax.fori_loop(0, W * 9, zb, None)

    def zs(j, _):
        pltpu.sync_copy(g0, acc.at[pl.ds(s * RPT + j * W, W)])
        return _
    lax.fori_loop(0, RPT // W, zs, None)

    plsc.subcore_barrier()

    lanes = lax.broadcasted_iota(jnp.int32, (16,), 0)
    hmask = lanes < 4
    mvec = jnp.where(hmask, mvr[...], 0.0)

    def fire(w, p):
        isl = src2v.at[w]
        dsl = dst2v.at[w]

        @pl.when(c == 0)
        def _():
            pltpu.async_copy(hx0.at[isl], GB[p], SG[p])
            pltpu.async_copy(at0.at[dsl], AB[p], SA[p])

        @pl.when(c == 1)
        def _():
            pltpu.async_copy(hx1.at[isl], GB[p], SG[p])
            pltpu.async_copy(at1.at[dsl], AB[p], SA[p])

    def wait_g(p):
        pltpu.make_async_copy(hx0.at[pl.ds(0, W)], GB[p], SG[p]).wait()
        pltpu.make_async_copy(at0.at[pl.ds(0, W)], AB[p], SA[p]).wait()

    def wait_s():
        pltpu.make_async_copy(hx0.at[pl.ds(0, W)], cb, ss).wait()

    def compute(p):
        gb, ab = GB[p], AB[p]

        def edge(r, _):
            asv = gb[r, pl.ds(HALF, 16)]
            adv = ab[r, pl.ds(0, 16)]
            pre = asv + adv
            ev = jnp.where(pre > 0.0, pre, 0.2 * pre)
            exv = jnp.where(hmask, jnp.exp(ev - mvec), 0.0)
            cb[r, pl.ds(HALF, 16)] = exv
            for hh in range(4):
                ebc = jnp.full((16,), exv[hh], jnp.float32)
                for k in (2 * hh, 2 * hh + 1):
                    cb[r, pl.ds(k * 16, 16)] = gb[r, pl.ds(k * 16, 16)] * ebc
            return _

        lax.fori_loop(0, W, edge, None)

    fire(0, 0)

    def wp_body(wp, _):
        for p in (0, 1):
            w = wp * 2 + p

            @pl.when(w + 1 < NWIN)
            def _():
                fire(w + 1, 1 - p)

            wait_g(p)

            @pl.when(w >= 1)
            def _():
                wait_s()

            compute(p)
            pltpu.async_copy(cb, acc.at[dst2v.at[w]], ss, add=True)
        return _

    lax.fori_loop(0, NWIN // 2, wp_body, None)
    wait_s()
    plsc.subcore_barrier()

    def rb(j, _):
        sl = pl.ds(s * RPT + j * W, W)

        @pl.when(c == 0)
        def _():
            pltpu.sync_copy(acc.at[sl], out0_h.at[sl])

        @pl.when(c == 1)
        def _():
            pltpu.sync_copy(acc.at[sl], out1_h.at[sl])
        return _
    lax.fori_loop(0, RPT // W, rb, None)


_sc_edge = functools.partial(
    pl.kernel,
    mesh=plsc.VectorSubcoreMesh(core_axis_name="c", subcore_axis_name="s"),
    compiler_params=pltpu.CompilerParams(use_tc_tiling_on_sc=False),
    out_type=[jax.ShapeDtypeStruct((NPAD, ROW), jnp.float32),
              jax.ShapeDtypeStruct((NPAD, ROW), jnp.float32)],
    scratch_types=[
        pltpu.VMEM((NWIN, W), jnp.int32),
        pltpu.VMEM((NWIN, W), jnp.int32),
        pltpu.VMEM((W, ROW), jnp.float32),
        pltpu.VMEM((W, ROW), jnp.float32),
        pltpu.VMEM((W, ROW), jnp.float32),
        pltpu.VMEM((W, 16), jnp.float32),
        pltpu.VMEM((W, 16), jnp.float32),
        pltpu.VMEM((16,), jnp.float32),
        pltpu.SemaphoreType.DMA,
        pltpu.SemaphoreType.DMA,
        pltpu.SemaphoreType.DMA,
        pltpu.SemaphoreType.DMA,
        pltpu.SemaphoreType.DMA,
        pltpu.VMEM_SHARED((NPAD, ROW), jnp.float32),
    ],
)(_sc_edge_body)


def _row_specs(n_args):
    return [pl.BlockSpec((BN, ROW), lambda i: (i, 0)) for _ in range(n_args)]


def _tc_outs():
    return (
        [jax.ShapeDtypeStruct((N, ROW), jnp.float32),
         jax.ShapeDtypeStruct((N, ROW), jnp.float32),
         jax.ShapeDtypeStruct((N, 16), jnp.float32),
         jax.ShapeDtypeStruct((N, 16), jnp.float32),
         jax.ShapeDtypeStruct((2, HEADS), jnp.float32)],
        [pl.BlockSpec((BN, ROW), lambda i: (i, 0)),
         pl.BlockSpec((BN, ROW), lambda i: (i, 0)),
         pl.BlockSpec((BN, 16), lambda i: (i, 0)),
         pl.BlockSpec((BN, 16), lambda i: (i, 0)),
         pl.BlockSpec((2, HEADS), lambda i: (0, 0))],
    )


def _full(shape):
    return pl.BlockSpec(shape, lambda i: tuple(0 for _ in shape))


def kernel(X, E, Ea, W1, as1, ad1, b1, W2, as2, ad2, b2,
           W3, as3, ad3, b3, W4, as4, ad4, b4):
    del Ea
    src = E[:, 0].astype(jnp.int32)
    dst = E[:, 1].astype(jnp.int32)
    src2 = jnp.pad(src.reshape(16, NWIN, W),
                   ((0, 0), (0, NWPAD - NWIN), (0, 0))).reshape(16 * NWPAD, W)
    dst2 = jnp.pad(dst.reshape(16, NWIN, W),
                   ((0, 0), (0, NWPAD - NWIN), (0, 0))).reshape(16 * NWPAD, W)

    atts = [(as1.reshape(HEADS, C), ad1.reshape(HEADS, C)),
            (as2.reshape(HEADS, C), ad2.reshape(HEADS, C)),
            (as3.reshape(HEADS, C), ad3.reshape(HEADS, C)),
            (as4.reshape(HEADS, C), ad4.reshape(HEADS, C))]
    Ws = [W1, W2, W3, W4]
    bs = [b1.reshape(1, F), b2.reshape(1, F), b3.reshape(1, F),
          b4.reshape(1, F)]

    outs, out_specs = _tc_outs()
    first = pl.pallas_call(
        _tc_first,
        grid=(GRID,),
        in_specs=[pl.BlockSpec((BN, 36), lambda i: (i, 0)),
                  _full((36, F)), _full((HEADS, C)), _full((HEADS, C))],
        out_specs=out_specs,
        out_shape=outs,
    )
    hx0, hx1, at0, at1, mo = first(X, W1, atts[0][0], atts[0][1])

    def m_vecs(mo):
        mv = jnp.maximum(mo[0] + mo[1], 0.0)
        m16 = jnp.concatenate([jnp.tile(mv[:4], 4), jnp.tile(mv[4:], 4)])
        return mv.reshape(1, HEADS), m16

    x3 = None
    for li in (1, 2, 3):
        mv, m16 = m_vecs(mo)
        agg0, agg1 = _sc_edge(hx0, hx1, at0, at1, src2, dst2, m16)
        outs, out_specs = _tc_outs()
        mid = pl.pallas_call(
            _tc_mid,
            grid=(GRID,),
            in_specs=([pl.BlockSpec((BN, ROW), lambda i: (i, 0)),
                       pl.BlockSpec((BN, ROW), lambda i: (i, 0)),
                       pl.BlockSpec((BN, ROW), lambda i: (i, 0)),
                       pl.BlockSpec((BN, ROW), lambda i: (i, 0)),
                       pl.BlockSpec((BN, 16), lambda i: (i, 0)),
                       pl.BlockSpec((BN, 16), lambda i: (i, 0)),
                       _full((1, HEADS)), _full((1, F)),
                       _full((F, F)), _full((HEADS, C)), _full((HEADS, C))]),
            out_specs=out_specs + [pl.BlockSpec((BN, F), lambda i: (i, 0))],
            out_shape=outs + [jax.ShapeDtypeStruct((N, F), jnp.float32)],
        )
        hx0, hx1, at0, at1, mo, x = mid(
            agg0, agg1, hx0, hx1, at0, at1, mv, bs[li - 1],
            Ws[li], atts[li][0], atts[li][1])
        if li == 3:
            x3 = x

    mv, m16 = m_vecs(mo)
    agg0, agg1 = _sc_edge(hx0, hx1, at0, at1, src2, dst2, m16)
    last = pl.pallas_call(
        _tc_last,
        grid=(GRID,),
        in_specs=([pl.BlockSpec((BN, ROW), lambda i: (i, 0)),
                   pl.BlockSpec((BN, ROW), lambda i: (i, 0)),
                   pl.BlockSpec((BN, ROW), lambda i: (i, 0)),
                   pl.BlockSpec((BN, ROW), lambda i: (i, 0)),
                   pl.BlockSpec((BN, 16), lambda i: (i, 0)),
                   pl.BlockSpec((BN, 16), lambda i: (i, 0)),
                   _full((1, HEADS)), _full((1, F))]),
        out_specs=pl.BlockSpec((BN, F), lambda i: (i, 0)),
        out_shape=jax.ShapeDtypeStruct((N, F), jnp.float32),
    )
    x4 = last(agg0, agg1, hx0, hx1, at0, at1, mv, bs[3])
    return (x3, x4)


# speedup vs baseline: 49.5891x; 1.0632x over previous
import functools

import jax
import jax.numpy as jnp
from jax import lax
from jax.experimental import pallas as pl
from jax.experimental.pallas import tpu as pltpu
from jax.experimental.pallas import tpu_sc as plsc

N = 10000
EDGES = 160000
HEADS = 8
C = 32
F = 256
HALF = 128
ROW = 144
BN = 1000
GRID = N // BN
W = 40
NWIN = 250
NWPAD = 256
EPT = EDGES // 16
NPAD = 10240
RPT = NPAD // 16

_HIGH = jax.lax.Precision.HIGHEST


def _dense_tail(x, w_ref, as_ref, ad_ref, hx0o, hx1o, at0o, at1o, moo):
    h = jnp.dot(x, w_ref[...], preferred_element_type=jnp.float32,
                precision=_HIGH)
    h3 = h.reshape(BN, HEADS, C)
    a_s = jnp.sum(h3 * as_ref[...][None], axis=-1)
    a_d = jnp.sum(h3 * ad_ref[...][None], axis=-1)
    pad = jnp.zeros((BN, ROW - HALF - 4), jnp.float32)
    hx0o[...] = jnp.concatenate([h[:, :HALF], a_s[:, :4], pad], axis=1)
    hx1o[...] = jnp.concatenate([h[:, HALF:], a_s[:, 4:], pad], axis=1)
    padd = jnp.zeros((BN, 12), jnp.float32)
    at0o[...] = jnp.concatenate([a_d[:, :4], padd], axis=1)
    at1o[...] = jnp.concatenate([a_d[:, 4:], padd], axis=1)
    blockm = jnp.stack([jnp.max(a_s, axis=0), jnp.max(a_d, axis=0)])

    @pl.when(pl.program_id(0) == 0)
    def _():
        moo[...] = jnp.full((2, HEADS), -1e30, jnp.float32)

    moo[...] = jnp.maximum(moo[...], blockm)


def _tc_first(x_ref, w_ref, as_ref, ad_ref, hx0o, hx1o, at0o, at1o, moo):
    _dense_tail(x_ref[...], w_ref, as_ref, ad_ref,
                hx0o, hx1o, at0o, at1o, moo)


def _normalize_half(o, hxp, atp, mh, bh):
    hh = hxp[:, :HALF]
    asv = hxp[:, HALF:HALF + 4]
    adv = atp[:, :4]
    pre = asv + adv
    ev = jnp.where(pre > 0.0, pre, 0.2 * pre)
    exs = jnp.exp(ev - mh[None, :])
    exb = jnp.reshape(jnp.broadcast_to(exs[:, :, None], (BN, 4, C)),
                      (BN, HALF))
    num = o[:, :HALF] + exb * hh
    den = o[:, HALF:HALF + 4] + exs + 1e-16
    denb = jnp.reshape(jnp.broadcast_to(den[:, :, None], (BN, 4, C)),
                       (BN, HALF))
    return num / denb + bh[None, :]


def _tc_mid(o0, o1, hx0, hx1, at0, at1, mv, bp, w_ref, as_ref, ad_ref,
            hx0o, hx1o, at0o, at1o, moo, xo):
    mvv = mv[0]
    x0 = _normalize_half(o0[...], hx0[...], at0, mvv[:4], bp[0, :HALF])
    x1 = _normalize_half(o1[...], hx1[...], at1, mvv[4:], bp[0, HALF:])
    x = jnp.concatenate([x0, x1], axis=1)
    xo[...] = x
    _dense_tail(x, w_ref, as_ref, ad_ref, hx0o, hx1o, at0o, at1o, moo)


def _tc_last(o0, o1, hx0, hx1, at0, at1, mv, bp, xo):
    mvv = mv[0]
    x0 = _normalize_half(o0[...], hx0[...], at0, mvv[:4], bp[0, :HALF])
    x1 = _normalize_half(o1[...], hx1[...], at1, mvv[4:], bp[0, HALF:])
    xo[...] = jnp.concatenate([x0, x1], axis=1)


def _sc_edge_body(hx0, hx1, at0, at1, src2_h, dst2_h, m_h,
                  out0_h, out1_h,
                  src2v, dst2v, g0, g1, cb, a0, a1, mvr,
                  sg0, sg1, sa0, sa1, ss, acc):
    c = lax.axis_index("c")
    s = lax.axis_index("s")
    z16 = jnp.zeros((16,), jnp.float32)
    GB, AB = (g0, g1), (a0, a1)
    SG, SA = (sg0, sg1), (sa0, sa1)

    pltpu.sync_copy(src2_h.at[pl.ds(s * NWPAD, NWIN)], src2v)
    pltpu.sync_copy(dst2_h.at[pl.ds(s * NWPAD, NWIN)], dst2v)
    pltpu.sync_copy(m_h.at[pl.ds(c * 16, 16)], mvr)

    def zb(i, _):
        g0[i // 9, pl.ds((i % 9) * 16, 16)] = z16
        return _
    lax.fori_loop(0, W * 9, zb, None)

    def zs(j, _):
        pltpu.sync_copy(g0, acc.at[pl.ds(s * RPT + j * W, W)])
        return _
    lax.fori_loop(0, RPT // W, zs, None)

    plsc.subcore_barrier()

    lanes = lax.broadcasted_iota(jnp.int32, (16,), 0)
    hmask = lanes < 4
    mvec = jnp.where(hmask, mvr[...], 0.0)

    def fire(w, p):
        isl = src2v.at[w]
        dsl = dst2v.at[w]

        @pl.when(c == 0)
        def _():
            pltpu.async_copy(hx0.at[isl], GB[p], SG[p])
            pltpu.async_copy(at0.at[dsl], AB[p], SA[p])

        @pl.when(c == 1)
        def _():
            pltpu.async_copy(hx1.at[isl], GB[p], SG[p])
            pltpu.async_copy(at1.at[dsl], AB[p], SA[p])

    def wait_g(p):
        pltpu.make_async_copy(hx0.at[pl.ds(0, W)], GB[p], SG[p]).wait()
        pltpu.make_async_copy(at0.at[pl.ds(0, W)], AB[p], SA[p]).wait()

    def wait_s():
        pltpu.make_async_copy(hx0.at[pl.ds(0, W)], cb, ss).wait()

    def compute(p):
        gb, ab = GB[p], AB[p]

        def edge2(t, _):
            rows = (t * 2, t * 2 + 1)
            exvs = []
            for r in rows:
                asv = gb[r, pl.ds(HALF, 16)]
                adv = ab[r, pl.ds(0, 16)]
                pre = asv + adv
                ev = jnp.where(pre > 0.0, pre, 0.2 * pre)
                exv = jnp.where(hmask, jnp.exp(ev - mvec), 0.0)
                cb[r, pl.ds(HALF, 16)] = exv
                exvs.append(exv)
            for r, exv in zip(rows, exvs):
                for hh in range(4):
                    ebc = jnp.full((16,), exv[hh], jnp.float32)
                    for k in (2 * hh, 2 * hh + 1):
                        cb[r, pl.ds(k * 16, 16)] = (
                            gb[r, pl.ds(k * 16, 16)] * ebc)
            return _

        lax.fori_loop(0, W // 2, edge2, None)

    fire(0, 0)

    def wp_body(wp, _):
        for p in (0, 1):
            w = wp * 2 + p

            @pl.when(w + 1 < NWIN)
            def _():
                fire(w + 1, 1 - p)

            wait_g(p)

            @pl.when(w >= 1)
            def _():
                wait_s()

            compute(p)
            pltpu.async_copy(cb, acc.at[dst2v.at[w]], ss, add=True)
        return _

    lax.fori_loop(0, NWIN // 2, wp_body, None)
    wait_s()
    plsc.subcore_barrier()

    def rb(j, _):
        sl = pl.ds(s * RPT + j * W, W)

        @pl.when(c == 0)
        def _():
            pltpu.sync_copy(acc.at[sl], out0_h.at[sl])

        @pl.when(c == 1)
        def _():
            pltpu.sync_copy(acc.at[sl], out1_h.at[sl])
        return _
    lax.fori_loop(0, RPT // W, rb, None)


_sc_edge = functools.partial(
    pl.kernel,
    mesh=plsc.VectorSubcoreMesh(core_axis_name="c", subcore_axis_name="s"),
    compiler_params=pltpu.CompilerParams(use_tc_tiling_on_sc=False),
    out_type=[jax.ShapeDtypeStruct((NPAD, ROW), jnp.float32),
              jax.ShapeDtypeStruct((NPAD, ROW), jnp.float32)],
    scratch_types=[
        pltpu.VMEM((NWIN, W), jnp.int32),
        pltpu.VMEM((NWIN, W), jnp.int32),
        pltpu.VMEM((W, ROW), jnp.float32),
        pltpu.VMEM((W, ROW), jnp.float32),
        pltpu.VMEM((W, ROW), jnp.float32),
        pltpu.VMEM((W, 16), jnp.float32),
        pltpu.VMEM((W, 16), jnp.float32),
        pltpu.VMEM((16,), jnp.float32),
        pltpu.SemaphoreType.DMA,
        pltpu.SemaphoreType.DMA,
        pltpu.SemaphoreType.DMA,
        pltpu.SemaphoreType.DMA,
        pltpu.SemaphoreType.DMA,
        pltpu.VMEM_SHARED((NPAD, ROW), jnp.float32),
    ],
)(_sc_edge_body)


def _row_specs(n_args):
    return [pl.BlockSpec((BN, ROW), lambda i: (i, 0)) for _ in range(n_args)]


def _tc_outs():
    return (
        [jax.ShapeDtypeStruct((N, ROW), jnp.float32),
         jax.ShapeDtypeStruct((N, ROW), jnp.float32),
         jax.ShapeDtypeStruct((N, 16), jnp.float32),
         jax.ShapeDtypeStruct((N, 16), jnp.float32),
         jax.ShapeDtypeStruct((2, HEADS), jnp.float32)],
        [pl.BlockSpec((BN, ROW), lambda i: (i, 0)),
         pl.BlockSpec((BN, ROW), lambda i: (i, 0)),
         pl.BlockSpec((BN, 16), lambda i: (i, 0)),
         pl.BlockSpec((BN, 16), lambda i: (i, 0)),
         pl.BlockSpec((2, HEADS), lambda i: (0, 0))],
    )


def _full(shape):
    return pl.BlockSpec(shape, lambda i: tuple(0 for _ in shape))


def kernel(X, E, Ea, W1, as1, ad1, b1, W2, as2, ad2, b2,
           W3, as3, ad3, b3, W4, as4, ad4, b4):
    del Ea
    src = E[:, 0].astype(jnp.int32)
    dst = E[:, 1].astype(jnp.int32)
    src2 = jnp.pad(src.reshape(16, NWIN, W),
                   ((0, 0), (0, NWPAD - NWIN), (0, 0))).reshape(16 * NWPAD, W)
    dst2 = jnp.pad(dst.reshape(16, NWIN, W),
                   ((0, 0), (0, NWPAD - NWIN), (0, 0))).reshape(16 * NWPAD, W)

    atts = [(as1.reshape(HEADS, C), ad1.reshape(HEADS, C)),
            (as2.reshape(HEADS, C), ad2.reshape(HEADS, C)),
            (as3.reshape(HEADS, C), ad3.reshape(HEADS, C)),
            (as4.reshape(HEADS, C), ad4.reshape(HEADS, C))]
    Ws = [W1, W2, W3, W4]
    bs = [b1.reshape(1, F), b2.reshape(1, F), b3.reshape(1, F),
          b4.reshape(1, F)]

    outs, out_specs = _tc_outs()
    first = pl.pallas_call(
        _tc_first,
        grid=(GRID,),
        in_specs=[pl.BlockSpec((BN, 36), lambda i: (i, 0)),
                  _full((36, F)), _full((HEADS, C)), _full((HEADS, C))],
        out_specs=out_specs,
        out_shape=outs,
    )
    hx0, hx1, at0, at1, mo = first(X, W1, atts[0][0], atts[0][1])

    def m_vecs(mo):
        mv = jnp.maximum(mo[0] + mo[1], 0.0)
        m16 = jnp.concatenate([jnp.tile(mv[:4], 4), jnp.tile(mv[4:], 4)])
        return mv.reshape(1, HEADS), m16

    x3 = None
    for li in (1, 2, 3):
        mv, m16 = m_vecs(mo)
        agg0, agg1 = _sc_edge(hx0, hx1, at0, at1, src2, dst2, m16)
        outs, out_specs = _tc_outs()
        mid = pl.pallas_call(
            _tc_mid,
            grid=(GRID,),
            in_specs=([pl.BlockSpec((BN, ROW), lambda i: (i, 0)),
                       pl.BlockSpec((BN, ROW), lambda i: (i, 0)),
                       pl.BlockSpec((BN, ROW), lambda i: (i, 0)),
                       pl.BlockSpec((BN, ROW), lambda i: (i, 0)),
                       pl.BlockSpec((BN, 16), lambda i: (i, 0)),
                       pl.BlockSpec((BN, 16), lambda i: (i, 0)),
                       _full((1, HEADS)), _full((1, F)),
                       _full((F, F)), _full((HEADS, C)), _full((HEADS, C))]),
            out_specs=out_specs + [pl.BlockSpec((BN, F), lambda i: (i, 0))],
            out_shape=outs + [jax.ShapeDtypeStruct((N, F), jnp.float32)],
        )
        hx0, hx1, at0, at1, mo, x = mid(
            agg0, agg1, hx0, hx1, at0, at1, mv, bs[li - 1],
            Ws[li], atts[li][0], atts[li][1])
        if li == 3:
            x3 = x

    mv, m16 = m_vecs(mo)
    agg0, agg1 = _sc_edge(hx0, hx1, at0, at1, src2, dst2, m16)
    last = pl.pallas_call(
        _tc_last,
        grid=(GRID,),
        in_specs=([pl.BlockSpec((BN, ROW), lambda i: (i, 0)),
                   pl.BlockSpec((BN, ROW), lambda i: (i, 0)),
                   pl.BlockSpec((BN, ROW), lambda i: (i, 0)),
                   pl.BlockSpec((BN, ROW), lambda i: (i, 0)),
                   pl.BlockSpec((BN, 16), lambda i: (i, 0)),
                   pl.BlockSpec((BN, 16), lambda i: (i, 0)),
                   _full((1, HEADS)), _full((1, F))]),
        out_specs=pl.BlockSpec((BN, F), lambda i: (i, 0)),
        out_shape=jax.ShapeDtypeStruct((N, F), jnp.float32),
    )
    x4 = last(agg0, agg1, hx0, hx1, at0, at1, mv, bs[3])
    return (x3, x4)
